# Initial kernel scaffold; baseline (speedup 1.0000x reference)
import jax
import jax.numpy as jnp
from jax.experimental import pallas as pl


def kernel(x, table, pos_table):
    raise NotImplementedError("write your pallas kernel here")


# speedup vs baseline: 1.3370x; 1.3370x over previous
import jax
import jax.numpy as jnp
from jax import lax
from jax.experimental import pallas as pl
from jax.experimental.pallas import tpu as pltpu
from jax.experimental.pallas import tpu_sc as plsc

VOCAB = 1000000
EMBED = 32
MAXLEN = 200
BATCH = 4096
SEQ = 200
N = BATCH * SEQ
NC, NS = 2, 16
NW = NC * NS
PER_W = N // NW
C = 3200
NCHUNK = PER_W // C


def _body(x_hbm, table_hbm, pos_hbm, out_hbm, idx_v, rows_v, pos_v, sem):
    wid = lax.axis_index("s") * NC + lax.axis_index("c")
    base = wid * PER_W
    pltpu.sync_copy(pos_hbm, pos_v)

    def chunk_body(ci, carry):
        off = base + ci * C
        pltpu.sync_copy(x_hbm.at[pl.ds(off, C)], idx_v)
        pltpu.async_copy(table_hbm.at[idx_v], rows_v, sem).wait()

        def add_body(r, carry2):
            p = lax.rem(r, MAXLEN)
            a = pos_v[p, pl.ds(0, 16)]
            b = pos_v[p, pl.ds(16, 16)]
            plsc.addupdate(rows_v.at[r, pl.ds(0, 16)], a)
            plsc.addupdate(rows_v.at[r, pl.ds(16, 16)], b)
            return carry2

        lax.fori_loop(0, C, add_body, 0)
        pltpu.sync_copy(rows_v, out_hbm.at[pl.ds(off, C)])
        return carry

    lax.fori_loop(0, NCHUNK, chunk_body, 0)


def kernel(x, table, pos_table):
    xf = x.reshape(N)
    k = pl.kernel(
        _body,
        out_type=jax.ShapeDtypeStruct((N, EMBED), jnp.float32),
        mesh=plsc.VectorSubcoreMesh(core_axis_name="c", subcore_axis_name="s"),
        compiler_params=pltpu.CompilerParams(use_tc_tiling_on_sc=False),
        scratch_types=[
            pltpu.VMEM((C,), jnp.int32),
            pltpu.VMEM((C, EMBED), jnp.float32),
            pltpu.VMEM((MAXLEN, EMBED), jnp.float32),
            pltpu.SemaphoreType.DMA,
        ],
    )
    out = k(xf, table, pos_table)
    return out.reshape(BATCH, SEQ, EMBED)
